# Initial kernel scaffold; baseline (speedup 1.0000x reference)
import jax
import jax.numpy as jnp
from jax.experimental import pallas as pl


def kernel(x, index):
    raise NotImplementedError("write your pallas kernel here")


# speedup vs baseline: 3.5254x; 3.5254x over previous
import jax
import jax.numpy as jnp
from jax import lax
from jax.experimental import pallas as pl
from jax.experimental.pallas import tpu as pltpu
from jax.experimental.pallas import tpu_sc as plsc

_B, _S, _D = 4, 8192, 1024
_NC, _NS = 2, 16
_NW = _NC * _NS
_RPW = _B * _S // _NW
_WPB = _S // _RPW
_C = 32
_NCHUNK = _RPW // _C
_LANES = 16


def _body(x_hbm, idx_hbm, out_hbm, idx_v, buf0, buf1,
          gsem0, gsem1, ssem0, ssem1):
    wid = lax.axis_index("s") * _NC + lax.axis_index("c")
    b = wid // _WPB
    seq_lo = (wid % _WPB) * _RPW
    out_lo = wid * _RPW

    pltpu.sync_copy(idx_hbm.at[pl.ds(seq_lo, _RPW)], idx_v)
    row_off = b * _S
    for j in range(_RPW // _LANES):
        sl = pl.ds(j * _LANES, _LANES)
        idx_v[sl] = idx_v[sl] + row_off

    bufs = (buf0, buf1)
    gsems = (gsem0, gsem1)
    ssems = (ssem0, ssem1)

    def _gather_desc(g, par):
        off = pl.multiple_of(g * _C, _C)
        src = x_hbm.at[idx_v.at[pl.ds(off, _C)]]
        return pltpu.make_async_copy(src, bufs[par], gsems[par])

    def _store_desc(g, par):
        dst = out_hbm.at[pl.ds(out_lo + g * _C, _C)]
        return pltpu.make_async_copy(bufs[par], dst, ssems[par])

    _gather_desc(0, 0).start()
    _gather_desc(1, 1).start()

    @pl.loop(0, _NCHUNK - 2, step=2)
    def _chunk_pair(g0):
        for par in range(2):
            g = g0 + par
            _gather_desc(g, par).wait()
            st = _store_desc(g, par)
            st.start()
            st.wait()
            _gather_desc(g + 2, par).start()

    for par in range(2):
        g = _NCHUNK - 2 + par
        _gather_desc(g, par).wait()
        _store_desc(g, par).start()
    for par in range(2):
        _store_desc(_NCHUNK - 2 + par, par).wait()


def _build():
    mesh = plsc.VectorSubcoreMesh(
        core_axis_name="c", subcore_axis_name="s",
        num_cores=_NC, num_subcores=_NS)
    return pl.kernel(
        _body,
        out_type=jax.ShapeDtypeStruct((_B * _S, _D), jnp.float32),
        mesh=mesh,
        scratch_types=[
            pltpu.VMEM((_RPW,), jnp.int32),
            pltpu.VMEM((_C, _D), jnp.float32),
            pltpu.VMEM((_C, _D), jnp.float32),
            pltpu.SemaphoreType.DMA,
            pltpu.SemaphoreType.DMA,
            pltpu.SemaphoreType.DMA,
            pltpu.SemaphoreType.DMA,
        ],
    )


def kernel(x, index):
    B, S, D = x.shape
    assert (B, S, D) == (_B, _S, _D)
    out = _build()(x.reshape(B * S, D), index.astype(jnp.int32))
    return out.reshape(B, S, D)
